# baseline (speedup 1.0000x reference)
import jax
import jax.numpy as jnp
from jax import lax
from jax.experimental import pallas as pl
from jax.experimental.pallas import tpu as pltpu
from jax.experimental.pallas import tpu_sc as plsc


_ROWS = 2048
_COLS = 8192
_NW = 32
_RPW = _ROWS // _NW
_CHR = 4
_NCH = _RPW // _CHR


def _sc_body(w_hbm, t_hbm, o_hbm, w_v, o_v, t_v):
    c = lax.axis_index("c")
    s_ax = lax.axis_index("s")
    wid = s_ax * 2 + c
    r0 = wid * _RPW

    pltpu.sync_copy(t_hbm.at[pl.ds(r0, _RPW)], t_v)
    for j in range(_RPW // 16):
        t16 = t_v[pl.ds(j * 16, 16)]
        t_v[pl.ds(j * 16, 16)] = 1.0 / (1.0 + jnp.exp(-t16))

    def chunk(ci, carry):
        row = r0 + ci * _CHR
        pltpu.sync_copy(w_hbm.at[pl.ds(row, _CHR), :], w_v)
        for r in range(_CHR):
            lrow = ci * _CHR + r
            blk = lrow // 16
            lane = lrow % 16
            t16 = t_v[pl.ds(blk * 16, 16)]
            starts = jnp.full((16, 1), lane, jnp.int32)
            s = lax.gather(
                t16, starts,
                lax.GatherDimensionNumbers(
                    offset_dims=(), collapsed_slice_dims=(0,),
                    start_index_map=(0,)),
                slice_sizes=(1,),
                mode=lax.GatherScatterMode.PROMISE_IN_BOUNDS)

            def inner(k, c2):
                v = w_v[r, pl.ds(k * 16, 16)]
                o_v[r, pl.ds(k * 16, 16)] = (
                    jnp.maximum(v - s, 0.0) + jnp.minimum(v + s, 0.0))
                return c2

            lax.fori_loop(0, _COLS // 16, inner, 0, unroll=8)
        pltpu.sync_copy(o_v, o_hbm.at[pl.ds(row, _CHR), :])
        return carry

    lax.fori_loop(0, _NCH, chunk, 0)


_sc_call = pl.kernel(
    _sc_body,
    out_type=jax.ShapeDtypeStruct((_ROWS, _COLS), jnp.float32),
    mesh=plsc.VectorSubcoreMesh(core_axis_name="c", subcore_axis_name="s"),
    scratch_types=[
        pltpu.VMEM((_CHR, _COLS), jnp.float32),
        pltpu.VMEM((_CHR, _COLS), jnp.float32),
        pltpu.VMEM((_RPW,), jnp.float32),
    ],
)


def kernel(weight, threshold, alpha):
    return _sc_call(weight, jnp.reshape(threshold, (_ROWS,)))


# speedup vs baseline: 6.3271x; 6.3271x over previous
import functools

import jax
import jax.numpy as jnp
from jax.experimental import pallas as pl
from jax.experimental.pallas import tpu as pltpu


_ROWS = 2048
_COLS = 8192
_CH = 32
_NBUF = 16


def _make_body(row0, nrows):
    num = nrows // _CH
    ngrp = num // _NBUF

    def body(w_hbm, t_hbm, o_hbm, w_buf, o_buf, t_v, in_sems, out_sems, t_sem):
        def in_copy(i, b):
            return pltpu.make_async_copy(
                w_hbm.at[pl.ds(row0 + i * _CH, _CH), :], w_buf.at[b],
                in_sems.at[b])

        def out_copy(i, b):
            return pltpu.make_async_copy(
                o_buf.at[b], o_hbm.at[pl.ds(i * _CH, _CH), :], out_sems.at[b])

        for b in range(_NBUF):
            in_copy(b, b).start()

        cp = pltpu.make_async_copy(t_hbm.at[pl.ds(row0, nrows), :], t_v, t_sem)
        cp.start()
        cp.wait()
        t_v[...] = jax.nn.sigmoid(t_v[...])

        for i in range(num):
            b = i % _NBUF
            in_copy(i, b).wait()
            if i >= _NBUF:
                out_copy(i - _NBUF, b).wait()
            w = w_buf[b]
            s = t_v[pl.ds(i * _CH, _CH), :]
            o_buf[b] = jnp.maximum(w - s, 0.0) + jnp.minimum(w + s, 0.0)
            out_copy(i, b).start()
            if i + _NBUF < num:
                in_copy(i + _NBUF, b).start()

        for i in range(num - _NBUF, num):
            out_copy(i, i % _NBUF).wait()

    return body


def _masked_rows(weight, threshold, row0, nrows):
    return pl.pallas_call(
        _make_body(row0, nrows),
        in_specs=[
            pl.BlockSpec(memory_space=pl.ANY),
            pl.BlockSpec(memory_space=pl.ANY),
        ],
        out_specs=pl.BlockSpec(memory_space=pl.ANY),
        out_shape=jax.ShapeDtypeStruct((nrows, _COLS), weight.dtype),
        scratch_shapes=[
            pltpu.VMEM((_NBUF, _CH, _COLS), jnp.float32),
            pltpu.VMEM((_NBUF, _CH, _COLS), jnp.float32),
            pltpu.VMEM((nrows, 1), jnp.float32),
            pltpu.SemaphoreType.DMA((_NBUF,)),
            pltpu.SemaphoreType.DMA((_NBUF,)),
            pltpu.SemaphoreType.DMA,
        ],
    )(weight, threshold)


def kernel(weight, threshold, alpha):
    return _masked_rows(weight, threshold, 0, _ROWS)
